# baseline (speedup 1.0000x reference)
import jax
import jax.numpy as jnp
from jax import lax
from jax.experimental import pallas as pl
from jax.experimental.pallas import tpu as pltpu
from jax.experimental.pallas import tpu_sc as plsc

_NC = 2
_NS = 16
_NW = _NC * _NS
_L = 16

_D = 64
_B = 4096 * 200
_CHUNK = 1024
_IDXW = 128
_NJ = _CHUNK // _IDXW
_PER_W = _B // _NW
_STEPS = _PER_W // _CHUNK


def _body(ids_hbm, table_hbm, out_hbm, idx_v, rows_v, sem):
    wid = lax.axis_index("s") * _NC + lax.axis_index("c")
    base = wid * _PER_W

    def step(g, _):
        row0 = pl.multiple_of((base + g * _CHUNK) // _IDXW, 8)
        pltpu.sync_copy(ids_hbm.at[pl.ds(row0, _NJ)], idx_v)
        for j in range(_NJ):
            for i in range(_IDXW // _L):
                sl = pl.ds(i * _L, _L)
                idx_v[j, sl] = idx_v[j, sl] + 1
        descs = [
            pltpu.async_copy(
                table_hbm.at[idx_v.at[j]],
                rows_v.at[pl.ds(j * _IDXW, _IDXW)],
                sem,
            )
            for j in range(_NJ)
        ]
        for d in descs:
            d.wait()
        out0 = pl.multiple_of(base + g * _CHUNK, 8)
        pltpu.sync_copy(rows_v, out_hbm.at[pl.ds(out0, _CHUNK)])
        return _

    lax.fori_loop(0, _STEPS, step, None)


def kernel(track_ids, embedding):
    b, s = track_ids.shape
    ids = track_ids.astype(jnp.int32).reshape(b * s // _IDXW, _IDXW)
    mesh = plsc.VectorSubcoreMesh(core_axis_name="c", subcore_axis_name="s")
    out = pl.kernel(
        _body,
        out_type=jax.ShapeDtypeStruct((_B, _D), jnp.float32),
        mesh=mesh,
        compiler_params=pltpu.CompilerParams(use_tc_tiling_on_sc=False),
        scratch_types=[
            pltpu.VMEM((_NJ, _IDXW), jnp.int32),
            pltpu.VMEM((_CHUNK, _D), jnp.float32),
            pltpu.SemaphoreType.DMA,
        ],
    )(ids, embedding)
    return out.reshape(b, s, _D)


# speedup vs baseline: 1.0210x; 1.0210x over previous
import jax
import jax.numpy as jnp
from jax import lax
from jax.experimental import pallas as pl
from jax.experimental.pallas import tpu as pltpu
from jax.experimental.pallas import tpu_sc as plsc

_NC = 2
_NS = 16
_NW = _NC * _NS
_L = 16

_D = 64
_B = 4096 * 200
_CHUNK = 512
_IDXW = 128
_NJ = _CHUNK // _IDXW
_PER_W = _B // _NW
_STEPS = _PER_W // _CHUNK


def _body(ids_hbm, table_hbm, out_hbm,
          idx0, idx1, rows0, rows1, gsem0, gsem1, wsem0, wsem1):
    wid = lax.axis_index("s") * _NC + lax.axis_index("c")
    base = wid * _PER_W

    def stage_idx(c, idxbuf):
        off = pl.multiple_of(base + c * _CHUNK, _CHUNK)
        pltpu.sync_copy(ids_hbm.at[pl.ds(off, _CHUNK)], idxbuf)
        for i in range(_CHUNK // _L):
            sl = pl.ds(i * _L, _L)
            idxbuf[sl] = idxbuf[sl] + 1

    def fire_gathers(idxbuf, rowsbuf, sem):
        for j in range(_NJ):
            sl = pl.ds(j * _IDXW, _IDXW)
            pltpu.async_copy(table_hbm.at[idxbuf.at[sl]], rowsbuf.at[sl], sem)

    def wait_gathers(rowsbuf, sem):
        pltpu.make_async_copy(table_hbm.at[pl.ds(0, _CHUNK)], rowsbuf, sem).wait()

    def fire_wb(c, rowsbuf, sem):
        off = pl.multiple_of(base + c * _CHUNK, _CHUNK)
        pltpu.async_copy(rowsbuf, out_hbm.at[pl.ds(off, _CHUNK)], sem)

    def wait_wb(rowsbuf, sem):
        pltpu.make_async_copy(rowsbuf, out_hbm.at[pl.ds(0, _CHUNK)], sem).wait()

    stage_idx(0, idx0)
    fire_gathers(idx0, rows0, gsem0)

    @pl.loop(0, _STEPS // 2)
    def _pair(k):
        a = 2 * k
        b = a + 1
        stage_idx(b, idx1)

        @pl.when(k > 0)
        def _():
            wait_wb(rows1, wsem1)

        fire_gathers(idx1, rows1, gsem1)
        wait_gathers(rows0, gsem0)
        fire_wb(a, rows0, wsem0)

        @pl.when(k < _STEPS // 2 - 1)
        def _():
            stage_idx(a + 2, idx0)
            wait_wb(rows0, wsem0)
            fire_gathers(idx0, rows0, gsem0)

        @pl.when(k == _STEPS // 2 - 1)
        def _():
            wait_wb(rows0, wsem0)

        wait_gathers(rows1, gsem1)
        fire_wb(b, rows1, wsem1)

    wait_wb(rows1, wsem1)


def kernel(track_ids, embedding):
    b, s = track_ids.shape
    ids = track_ids.astype(jnp.int32).reshape(b * s)
    mesh = plsc.VectorSubcoreMesh(core_axis_name="c", subcore_axis_name="s")
    out = pl.kernel(
        _body,
        out_type=jax.ShapeDtypeStruct((_B, _D), jnp.float32),
        mesh=mesh,
        compiler_params=pltpu.CompilerParams(use_tc_tiling_on_sc=False),
        scratch_types=[
            pltpu.VMEM((_CHUNK,), jnp.int32),
            pltpu.VMEM((_CHUNK,), jnp.int32),
            pltpu.VMEM((_CHUNK, _D), jnp.float32),
            pltpu.VMEM((_CHUNK, _D), jnp.float32),
            pltpu.SemaphoreType.DMA,
            pltpu.SemaphoreType.DMA,
            pltpu.SemaphoreType.DMA,
            pltpu.SemaphoreType.DMA,
        ],
    )(ids, embedding)
    return out.reshape(b, s, _D)
